# baseline (speedup 1.0000x reference)
import functools

import jax
import jax.numpy as jnp
import numpy as np
from jax.experimental import pallas as pl
from jax.experimental.pallas import tpu as pltpu

MAX_LEN = 2048
D_PE = 16
N_TS = D_PE // 2
HIGHEST = jax.lax.Precision.HIGHEST
DN_T = (((1,), (1,)), ((), ()))


def _np_tables(V):
    ts = (MAX_LEN ** np.linspace(0.0, 1.0, N_TS)).astype(np.float32)
    ts_row = np.repeat(ts, 16).reshape(1, N_TS * 16).astype(np.float32)
    scaled = np.arange(V, dtype=np.float32)[:, None] / ts[None, :]
    var_pe = np.concatenate([np.sin(scaled), np.cos(scaled)], axis=1)
    ones_v16 = np.ones((V, 16), np.float32)
    eye16 = np.eye(16, dtype=np.float32)
    return ts_row, var_pe.astype(np.float32), ones_v16, eye16


def _np_selpat(rows):
    return (np.arange(rows)[None, :] % 16 == np.arange(16)[:, None]).astype(
        np.float32)


def _seft_body(src_ref, times_ref, static_ref, tsrow_ref, varpe_ref,
               onesv_ref, eye_ref, selpat_ref, wv_ref, bv_ref, wsum_ref,
               bmap_ref, wemb_ref, bemb_ref, wm1_ref, bm1_ref, wm2_ref,
               bm2_ref, out_ref, acc_sin, acc_cos, colcnt, colfea):
    i = pl.program_id(0)
    f32 = jnp.float32
    bf16 = jnp.bfloat16

    @pl.when(i == 0)
    def _init():
        acc_sin[...] = jnp.zeros_like(acc_sin)
        acc_cos[...] = jnp.zeros_like(acc_cos)
        colcnt[...] = jnp.zeros_like(colcnt)
        colfea[...] = jnp.zeros_like(colfea)

    x = src_ref[...]
    rb, w = x.shape
    v = w // 2
    tb3 = rb // 16
    fea = x[:, :v]
    obs = (fea != 0.0).astype(bf16)
    colcnt[...] += jax.lax.dot_general(
        selpat_ref[...].astype(bf16), obs, (((1,), (0,)), ((), ())),
        preferred_element_type=f32)
    colfea[...] += jnp.sum(fea.reshape(tb3, 16, v), axis=0)
    z = jax.lax.dot_general(
        obs, onesv_ref[...].astype(bf16), (((1,), (0,)), ((), ())),
        preferred_element_type=f32)
    z3 = z.reshape(tb3, 16, 16)
    rowcnt = jnp.sum(z3 * eye_ref[...][None, :, :], axis=1)

    tb = times_ref[...]
    t_big = jnp.concatenate([tb] * N_TS, axis=1) / tsrow_ref[...]
    rc8 = jnp.concatenate([rowcnt] * N_TS, axis=1)
    acc_sin[...] += jnp.sum(jnp.sin(t_big) * rc8, axis=0, keepdims=True)
    acc_cos[...] += jnp.sum(jnp.cos(t_big) * rc8, axis=0, keepdims=True)

    @pl.when(i == pl.num_programs(0) - 1)
    def _epilogue():
        cc = colcnt[...]
        count = jnp.sum(cc, axis=1, keepdims=True)
        sumfea = jnp.sum(colfea[...], axis=1, keepdims=True)
        denom = jnp.maximum(count, 1.0)
        sum_var = jnp.dot(cc, varpe_ref[...],
                          preferred_element_type=jnp.float32,
                          precision=HIGHEST)

        asin = acc_sin[...]
        acosv = acc_cos[...]
        rows = [asin[0:1, 16 * d:16 * (d + 1)] for d in range(N_TS)]
        rows += [acosv[0:1, 16 * d:16 * (d + 1)] for d in range(N_TS)]
        spe_t = jnp.concatenate(rows, axis=0)

        wsum = wsum_ref[:, :3 * D_PE] + wsum_ref[:, 3 * D_PE:]
        w_pe = wsum[:, 0:16]
        w_val = wsum[:, 16:32]
        w_var = wsum[:, 32:48]
        term_pe = jax.lax.dot_general(
            spe_t, w_pe, (((0,), (1,)), ((), ())),
            preferred_element_type=jnp.float32, precision=HIGHEST)
        sum_val = (jnp.dot(sumfea, wv_ref[...].reshape(1, 16),
                           preferred_element_type=jnp.float32,
                           precision=HIGHEST)
                   + jnp.dot(count, bv_ref[...].reshape(1, 16),
                             preferred_element_type=jnp.float32,
                             precision=HIGHEST))
        term_val = jax.lax.dot_general(
            sum_val, w_val, DN_T,
            preferred_element_type=jnp.float32, precision=HIGHEST)
        term_var = jax.lax.dot_general(
            sum_var, w_var, DN_T,
            preferred_element_type=jnp.float32, precision=HIGHEST)

        raw = term_pe + term_val + term_var
        out128 = jnp.where(count > 0, raw / denom, 0.0) + bmap_ref[...]
        emb = jax.lax.dot_general(
            static_ref[...], wemb_ref[...], DN_T,
            preferred_element_type=jnp.float32, precision=HIGHEST)
        emb = emb + bemb_ref[...]
        cat = jnp.concatenate([out128, emb], axis=1)
        h = jnp.maximum(
            jax.lax.dot_general(cat, wm1_ref[...], DN_T,
                                preferred_element_type=jnp.float32,
                                precision=HIGHEST) + bm1_ref[...], 0.0)
        out_ref[...] = jax.lax.dot_general(
            h, wm2_ref[...], DN_T, preferred_element_type=jnp.float32,
            precision=HIGHEST) + bm2_ref[...]


@functools.partial(jax.jit, static_argnames=())
def _seft(src, static, times, W_value, b_value, W_map, b_map, W_emb, b_emb,
          W_mlp1, b_mlp1, W_mlp2, b_mlp2):
    T, B = src.shape[0], src.shape[1]
    V = src.shape[2] // 2
    TB = 256
    grid = T // TB
    src_rows = src.reshape(T * B, 2 * V)

    ts_row, var_pe, ones_v16, eye16 = map(jnp.asarray, _np_tables(V))
    selpat = jnp.asarray(_np_selpat(TB * B))

    full = lambda shape: pl.BlockSpec(shape, lambda i: tuple(0 for _ in shape))
    operands = (
        src_rows, times, static, ts_row, var_pe, ones_v16, eye16, selpat,
        W_value.reshape(1, 16), b_value.reshape(1, 16),
        W_map, b_map.reshape(1, -1),
        W_emb, b_emb.reshape(1, -1),
        W_mlp1, b_mlp1.reshape(1, -1),
        W_mlp2, b_mlp2.reshape(1, -1),
    )
    in_specs = [
        pl.BlockSpec((TB * B, 2 * V), lambda i: (i, 0)),
        pl.BlockSpec((TB, B), lambda i: (i, 0)),
    ] + [full(op.shape) for op in operands[2:]]

    return pl.pallas_call(
        _seft_body,
        grid=(grid,),
        in_specs=in_specs,
        out_specs=pl.BlockSpec((B, 2), lambda i: (0, 0)),
        out_shape=jax.ShapeDtypeStruct((B, 2), jnp.float32),
        scratch_shapes=[
            pltpu.VMEM((1, 128), jnp.float32),
            pltpu.VMEM((1, 128), jnp.float32),
            pltpu.VMEM((B, V), jnp.float32),
            pltpu.VMEM((B, V), jnp.float32),
        ],
        compiler_params=pltpu.CompilerParams(
            dimension_semantics=("arbitrary",)),
    )(*operands)


def kernel(src, static, times, lengths, W_value, b_value, W_map, b_map,
           W_emb, b_emb, W_mlp1, b_mlp1, W_mlp2, b_mlp2):
    del lengths
    return _seft(src, static, times, W_value, b_value, W_map, b_map,
                 W_emb, b_emb, W_mlp1, b_mlp1, W_mlp2, b_mlp2)


# speedup vs baseline: 1.2666x; 1.2666x over previous
import functools

import jax
import jax.numpy as jnp
import numpy as np
from jax.experimental import pallas as pl
from jax.experimental.pallas import tpu as pltpu

MAX_LEN = 2048
D_PE = 16
N_TS = D_PE // 2
HIGHEST = jax.lax.Precision.HIGHEST
DN_T = (((1,), (1,)), ((), ()))


def _np_tables(V):
    ts = (MAX_LEN ** np.linspace(0.0, 1.0, N_TS)).astype(np.float32)
    ts_row = np.repeat(ts, 16).reshape(1, N_TS * 16).astype(np.float32)
    scaled = np.arange(V, dtype=np.float32)[:, None] / ts[None, :]
    var_pe = np.concatenate([np.sin(scaled), np.cos(scaled)], axis=1)
    ones_v16 = np.ones((V, 16), np.float32)
    eye16 = np.eye(16, dtype=np.float32)
    return ts_row, var_pe.astype(np.float32), ones_v16, eye16


def _np_selpat(rows):
    return (np.arange(rows)[None, :] % 16 == np.arange(16)[:, None]).astype(
        np.float32)


def _seft_body(src_ref, times_ref, static_ref, tsrow_ref, varpe_ref,
               onesv_ref, eye_ref, selpat_ref, wv_ref, bv_ref, wsum_ref,
               bmap_ref, wemb_ref, bemb_ref, wm1_ref, bm1_ref, wm2_ref,
               bm2_ref, out_ref, acc_sin, acc_cos, colcnt, colfea):
    i = pl.program_id(0)
    f32 = jnp.float32
    bf16 = jnp.bfloat16

    @pl.when(i == 0)
    def _init():
        acc_sin[...] = jnp.zeros_like(acc_sin)
        acc_cos[...] = jnp.zeros_like(acc_cos)
        colcnt[...] = jnp.zeros_like(colcnt)
        colfea[...] = jnp.zeros_like(colfea)

    x3 = src_ref[...]
    tb3, _, w = x3.shape
    v = w // 2
    rb = tb3 * 16
    x = x3.reshape(rb, w)
    fea = x[:, :v]
    obs = (fea != 0.0).astype(bf16)
    colcnt[...] += jax.lax.dot_general(
        selpat_ref[...].astype(bf16), obs, (((1,), (0,)), ((), ())),
        preferred_element_type=f32)
    colfea[...] += jnp.sum(fea.reshape(tb3, 16, v), axis=0)
    z = jax.lax.dot_general(
        obs, onesv_ref[...].astype(bf16), (((1,), (0,)), ((), ())),
        preferred_element_type=f32)
    z3 = z.reshape(tb3, 16, 16)
    rowcnt = jnp.sum(z3 * eye_ref[...][None, :, :], axis=1)

    tb = times_ref[...]
    t_big = jnp.concatenate([tb] * N_TS, axis=1) / tsrow_ref[...]
    rc8 = jnp.concatenate([rowcnt] * N_TS, axis=1)
    acc_sin[...] += jnp.sum(jnp.sin(t_big) * rc8, axis=0, keepdims=True)
    acc_cos[...] += jnp.sum(jnp.cos(t_big) * rc8, axis=0, keepdims=True)

    @pl.when(i == pl.num_programs(0) - 1)
    def _epilogue():
        cc = colcnt[...]
        count = jnp.sum(cc, axis=1, keepdims=True)
        sumfea = jnp.sum(colfea[...], axis=1, keepdims=True)
        denom = jnp.maximum(count, 1.0)
        sum_var = jnp.dot(cc, varpe_ref[...],
                          preferred_element_type=jnp.float32,
                          precision=HIGHEST)

        asin = acc_sin[...]
        acosv = acc_cos[...]
        rows = [asin[0:1, 16 * d:16 * (d + 1)] for d in range(N_TS)]
        rows += [acosv[0:1, 16 * d:16 * (d + 1)] for d in range(N_TS)]
        spe_t = jnp.concatenate(rows, axis=0)

        wsum = wsum_ref[:, :3 * D_PE] + wsum_ref[:, 3 * D_PE:]
        w_pe = wsum[:, 0:16]
        w_val = wsum[:, 16:32]
        w_var = wsum[:, 32:48]
        term_pe = jax.lax.dot_general(
            spe_t, w_pe, (((0,), (1,)), ((), ())),
            preferred_element_type=jnp.float32, precision=HIGHEST)
        sum_val = (jnp.dot(sumfea, wv_ref[...].reshape(1, 16),
                           preferred_element_type=jnp.float32,
                           precision=HIGHEST)
                   + jnp.dot(count, bv_ref[...].reshape(1, 16),
                             preferred_element_type=jnp.float32,
                             precision=HIGHEST))
        term_val = jax.lax.dot_general(
            sum_val, w_val, DN_T,
            preferred_element_type=jnp.float32, precision=HIGHEST)
        term_var = jax.lax.dot_general(
            sum_var, w_var, DN_T,
            preferred_element_type=jnp.float32, precision=HIGHEST)

        raw = term_pe + term_val + term_var
        out128 = jnp.where(count > 0, raw / denom, 0.0) + bmap_ref[...]
        emb = jax.lax.dot_general(
            static_ref[...], wemb_ref[...], DN_T,
            preferred_element_type=jnp.float32, precision=HIGHEST)
        emb = emb + bemb_ref[...]
        cat = jnp.concatenate([out128, emb], axis=1)
        h = jnp.maximum(
            jax.lax.dot_general(cat, wm1_ref[...], DN_T,
                                preferred_element_type=jnp.float32,
                                precision=HIGHEST) + bm1_ref[...], 0.0)
        out_ref[...] = jax.lax.dot_general(
            h, wm2_ref[...], DN_T, preferred_element_type=jnp.float32,
            precision=HIGHEST) + bm2_ref[...]


@functools.partial(jax.jit, static_argnames=())
def _seft(src, static, times, W_value, b_value, W_map, b_map, W_emb, b_emb,
          W_mlp1, b_mlp1, W_mlp2, b_mlp2):
    T, B = src.shape[0], src.shape[1]
    V = src.shape[2] // 2
    TB = 256
    grid = T // TB

    ts_row, var_pe, ones_v16, eye16 = map(jnp.asarray, _np_tables(V))
    selpat = jnp.asarray(_np_selpat(TB * B))

    full = lambda shape: pl.BlockSpec(shape, lambda i: tuple(0 for _ in shape))
    operands = (
        src, times, static, ts_row, var_pe, ones_v16, eye16, selpat,
        W_value.reshape(1, 16), b_value.reshape(1, 16),
        W_map, b_map.reshape(1, -1),
        W_emb, b_emb.reshape(1, -1),
        W_mlp1, b_mlp1.reshape(1, -1),
        W_mlp2, b_mlp2.reshape(1, -1),
    )
    in_specs = [
        pl.BlockSpec((TB, B, 2 * V), lambda i: (i, 0, 0)),
        pl.BlockSpec((TB, B), lambda i: (i, 0)),
    ] + [full(op.shape) for op in operands[2:]]

    return pl.pallas_call(
        _seft_body,
        grid=(grid,),
        in_specs=in_specs,
        out_specs=pl.BlockSpec((B, 2), lambda i: (0, 0)),
        out_shape=jax.ShapeDtypeStruct((B, 2), jnp.float32),
        scratch_shapes=[
            pltpu.VMEM((1, 128), jnp.float32),
            pltpu.VMEM((1, 128), jnp.float32),
            pltpu.VMEM((B, V), jnp.float32),
            pltpu.VMEM((B, V), jnp.float32),
        ],
        compiler_params=pltpu.CompilerParams(
            dimension_semantics=("arbitrary",)),
    )(*operands)


def kernel(src, static, times, lengths, W_value, b_value, W_map, b_map,
           W_emb, b_emb, W_mlp1, b_mlp1, W_mlp2, b_mlp2):
    del lengths
    return _seft(src, static, times, W_value, b_value, W_map, b_map,
                 W_emb, b_emb, W_mlp1, b_mlp1, W_mlp2, b_mlp2)


# speedup vs baseline: 3.8214x; 3.0170x over previous
import functools

import jax
import jax.numpy as jnp
import numpy as np
from jax.experimental import pallas as pl
from jax.experimental.pallas import tpu as pltpu

MAX_LEN = 2048
D_PE = 16
N_TS = D_PE // 2
HIGHEST = jax.lax.Precision.HIGHEST
DN_T = (((1,), (1,)), ((), ()))


def _np_tables(V):
    ts = (MAX_LEN ** np.linspace(0.0, 1.0, N_TS)).astype(np.float32)
    ts_col = np.repeat(ts, 16).reshape(N_TS * 16, 1).astype(np.float32)
    scaled = np.arange(V, dtype=np.float32)[:, None] / ts[None, :]
    var_pe = np.concatenate([np.sin(scaled), np.cos(scaled)], axis=1)
    eye128 = np.eye(128, dtype=np.float32)
    return ts_col, var_pe.astype(np.float32), eye128


def _seft_body(src_ref, times_ref, static_ref, tscol_ref, varpe_ref,
               eye_ref, wv_ref, bv_ref, wmapT_ref, bmap_ref,
               wemb_ref, bemb_ref, wm1_ref, bm1_ref, wm2_ref, bm2_ref,
               out_ref, acc_sin, acc_cos, colcnt_l, colfea_l):
    i = pl.program_id(0)
    f32 = jnp.float32

    @pl.when(i == 0)
    def _init():
        acc_sin[...] = jnp.zeros_like(acc_sin)
        acc_cos[...] = jnp.zeros_like(acc_cos)
        colcnt_l[...] = jnp.zeros_like(colcnt_l)
        colfea_l[...] = jnp.zeros_like(colfea_l)

    x = src_ref[...]
    v = x.shape[1] // 2
    fea = x[:, :v, :]
    mask = (fea != 0.0).astype(f32)
    colcnt_l[...] += mask
    colfea_l[...] += fea
    rowcnt = jnp.sum(mask, axis=1)

    tb = times_ref[...]
    t_big = jnp.concatenate([tb] * N_TS, axis=0) / tscol_ref[...]
    rc8 = jnp.concatenate([rowcnt] * N_TS, axis=0)
    acc_sin[...] += jnp.sin(t_big) * rc8
    acc_cos[...] += jnp.cos(t_big) * rc8

    @pl.when(i == pl.num_programs(0) - 1)
    def _epilogue():
        cc = jnp.sum(colcnt_l[...], axis=2)
        cf = jnp.sum(colfea_l[...], axis=2)
        count = jnp.sum(cc, axis=1, keepdims=True)
        sumfea = jnp.sum(cf, axis=1, keepdims=True)
        denom = jnp.maximum(count, 1.0)
        sum_var = jnp.dot(cc, varpe_ref[...],
                          preferred_element_type=f32, precision=HIGHEST)

        acc2 = jnp.concatenate(
            [jnp.sum(acc_sin[...], axis=1, keepdims=True),
             jnp.sum(acc_cos[...], axis=1, keepdims=True)], axis=1)
        accr = jax.lax.dot_general(
            acc2, eye_ref[...], (((0,), (0,)), ((), ())),
            preferred_element_type=f32, precision=HIGHEST)
        rows = [accr[0:1, 16 * d:16 * (d + 1)] for d in range(N_TS)]
        rows += [accr[1:2, 16 * d:16 * (d + 1)] for d in range(N_TS)]
        spe_t = jnp.concatenate(rows, axis=0)

        wsum = wmapT_ref[0:3 * D_PE, :] + wmapT_ref[3 * D_PE:, :]
        w_pe = wsum[0:16, :]
        w_val = wsum[16:32, :]
        w_var = wsum[32:48, :]
        term_pe = jax.lax.dot_general(
            spe_t, w_pe, (((0,), (0,)), ((), ())),
            preferred_element_type=f32, precision=HIGHEST)
        sum_val = (jnp.dot(sumfea, wv_ref[...],
                           preferred_element_type=f32, precision=HIGHEST)
                   + jnp.dot(count, bv_ref[...],
                             preferred_element_type=f32, precision=HIGHEST))
        term_val = jnp.dot(sum_val, w_val,
                           preferred_element_type=f32, precision=HIGHEST)
        term_var = jnp.dot(sum_var, w_var,
                           preferred_element_type=f32, precision=HIGHEST)

        raw = term_pe + term_val + term_var
        out128 = jnp.where(count > 0, raw / denom, 0.0) + bmap_ref[...]
        emb = jax.lax.dot_general(
            static_ref[...], wemb_ref[...], DN_T,
            preferred_element_type=f32, precision=HIGHEST) + bemb_ref[...]
        cat = jnp.concatenate([out128, emb], axis=1)
        h = jnp.maximum(
            jax.lax.dot_general(cat, wm1_ref[...], DN_T,
                                preferred_element_type=f32,
                                precision=HIGHEST) + bm1_ref[...], 0.0)
        out_ref[...] = jax.lax.dot_general(
            h, wm2_ref[...], DN_T, preferred_element_type=f32,
            precision=HIGHEST) + bm2_ref[...]


@functools.partial(jax.jit, static_argnames=())
def _seft(src, static, times, W_value, b_value, W_map, b_map, W_emb, b_emb,
          W_mlp1, b_mlp1, W_mlp2, b_mlp2):
    T, B = src.shape[0], src.shape[1]
    V = src.shape[2] // 2
    TBT = 256
    grid = T // TBT

    srcT = jnp.transpose(src, (1, 2, 0))
    timesT = jnp.transpose(times)
    wmapT = W_map.T

    ts_col, var_pe, eye128 = map(jnp.asarray, _np_tables(V))

    full = lambda shape: pl.BlockSpec(shape, lambda i: tuple(0 for _ in shape))
    operands = (
        srcT, timesT, static, ts_col, var_pe, eye128,
        W_value.reshape(1, 16), b_value.reshape(1, 16),
        wmapT, b_map.reshape(1, -1),
        W_emb, b_emb.reshape(1, -1),
        W_mlp1, b_mlp1.reshape(1, -1),
        W_mlp2, b_mlp2.reshape(1, -1),
    )
    in_specs = [
        pl.BlockSpec((B, 2 * V, TBT), lambda i: (0, 0, i)),
        pl.BlockSpec((B, TBT), lambda i: (0, i)),
    ] + [full(op.shape) for op in operands[2:]]

    return pl.pallas_call(
        _seft_body,
        grid=(grid,),
        in_specs=in_specs,
        out_specs=pl.BlockSpec((B, 2), lambda i: (0, 0)),
        out_shape=jax.ShapeDtypeStruct((B, 2), jnp.float32),
        scratch_shapes=[
            pltpu.VMEM((8 * D_PE, TBT), jnp.float32),
            pltpu.VMEM((8 * D_PE, TBT), jnp.float32),
            pltpu.VMEM((B, V, TBT), jnp.float32),
            pltpu.VMEM((B, V, TBT), jnp.float32),
        ],
        compiler_params=pltpu.CompilerParams(
            dimension_semantics=("arbitrary",)),
    )(*operands)


def kernel(src, static, times, lengths, W_value, b_value, W_map, b_map,
           W_emb, b_emb, W_mlp1, b_mlp1, W_mlp2, b_mlp2):
    del lengths
    return _seft(src, static, times, W_value, b_value, W_map, b_map,
                 W_emb, b_emb, W_mlp1, b_mlp1, W_mlp2, b_mlp2)


# speedup vs baseline: 4.1936x; 1.0974x over previous
import functools

import jax
import jax.numpy as jnp
import numpy as np
from jax.experimental import pallas as pl
from jax.experimental.pallas import tpu as pltpu

MAX_LEN = 2048
D_PE = 16
N_TS = D_PE // 2
HIGHEST = jax.lax.Precision.HIGHEST
DN_T = (((1,), (1,)), ((), ()))


def _np_tables(V):
    ts = (MAX_LEN ** np.linspace(0.0, 1.0, N_TS)).astype(np.float32)
    ts_col = np.repeat(ts, 16).reshape(N_TS * 16, 1).astype(np.float32)
    scaled = np.arange(V, dtype=np.float32)[:, None] / ts[None, :]
    var_pe = np.concatenate([np.sin(scaled), np.cos(scaled)], axis=1)
    eye128 = np.eye(128, dtype=np.float32)
    return ts_col, var_pe.astype(np.float32), eye128


def _seft_body(src_ref, times_ref, static_ref, tscol_ref, varpe_ref,
               eye_ref, wv_ref, bv_ref, wmapT_ref, bmap_ref,
               wemb_ref, bemb_ref, wm1_ref, bm1_ref, wm2_ref, bm2_ref,
               out_ref, acc_sin, acc_cos, colcnt_l, colfea_l):
    i = pl.program_id(0)
    f32 = jnp.float32

    @pl.when(i == 0)
    def _init():
        acc_sin[...] = jnp.zeros_like(acc_sin)
        acc_cos[...] = jnp.zeros_like(acc_cos)
        colcnt_l[...] = jnp.zeros_like(colcnt_l)
        colfea_l[...] = jnp.zeros_like(colfea_l)

    x = src_ref[...]
    v = x.shape[1] // 2
    fea = x[:, :v, :]
    mask = (fea != 0.0).astype(f32)
    colcnt_l[...] += mask
    colfea_l[...] += fea
    rowcnt = jnp.sum(mask, axis=1)

    tb = times_ref[...]
    t_big = jnp.concatenate([tb] * N_TS, axis=0) / tscol_ref[...]
    rc8 = jnp.concatenate([rowcnt] * N_TS, axis=0)
    acc_sin[...] += jnp.sin(t_big) * rc8
    acc_cos[...] += jnp.cos(t_big) * rc8

    @pl.when(i == pl.num_programs(0) - 1)
    def _epilogue():
        cc = jnp.sum(colcnt_l[...], axis=2)
        cf = jnp.sum(colfea_l[...], axis=2)
        count = jnp.sum(cc, axis=1, keepdims=True)
        sumfea = jnp.sum(cf, axis=1, keepdims=True)
        denom = jnp.maximum(count, 1.0)
        sum_var = jnp.dot(cc, varpe_ref[...],
                          preferred_element_type=f32, precision=HIGHEST)

        acc2 = jnp.concatenate(
            [jnp.sum(acc_sin[...], axis=1, keepdims=True),
             jnp.sum(acc_cos[...], axis=1, keepdims=True)], axis=1)
        accr = jax.lax.dot_general(
            acc2, eye_ref[...], (((0,), (0,)), ((), ())),
            preferred_element_type=f32, precision=HIGHEST)
        rows = [accr[0:1, 16 * d:16 * (d + 1)] for d in range(N_TS)]
        rows += [accr[1:2, 16 * d:16 * (d + 1)] for d in range(N_TS)]
        spe_t = jnp.concatenate(rows, axis=0)

        wsum = wmapT_ref[0:3 * D_PE, :] + wmapT_ref[3 * D_PE:, :]
        w_pe = wsum[0:16, :]
        w_val = wsum[16:32, :]
        w_var = wsum[32:48, :]
        term_pe = jax.lax.dot_general(
            spe_t, w_pe, (((0,), (0,)), ((), ())),
            preferred_element_type=f32, precision=HIGHEST)
        sum_val = (jnp.dot(sumfea, wv_ref[...],
                           preferred_element_type=f32, precision=HIGHEST)
                   + jnp.dot(count, bv_ref[...],
                             preferred_element_type=f32, precision=HIGHEST))
        term_val = jnp.dot(sum_val, w_val,
                           preferred_element_type=f32, precision=HIGHEST)
        term_var = jnp.dot(sum_var, w_var,
                           preferred_element_type=f32, precision=HIGHEST)

        raw = term_pe + term_val + term_var
        out128 = jnp.where(count > 0, raw / denom, 0.0) + bmap_ref[...]
        emb = jax.lax.dot_general(
            static_ref[...], wemb_ref[...], DN_T,
            preferred_element_type=f32, precision=HIGHEST) + bemb_ref[...]
        cat = jnp.concatenate([out128, emb], axis=1)
        h = jnp.maximum(
            jax.lax.dot_general(cat, wm1_ref[...], DN_T,
                                preferred_element_type=f32,
                                precision=HIGHEST) + bm1_ref[...], 0.0)
        out_ref[...] = jax.lax.dot_general(
            h, wm2_ref[...], DN_T, preferred_element_type=f32,
            precision=HIGHEST) + bm2_ref[...]


@functools.partial(jax.jit, static_argnames=())
def _seft(src, static, times, W_value, b_value, W_map, b_map, W_emb, b_emb,
          W_mlp1, b_mlp1, W_mlp2, b_mlp2):
    T, B = src.shape[0], src.shape[1]
    V = src.shape[2] // 2
    TBT = 512
    grid = T // TBT

    srcT = jnp.transpose(src, (1, 2, 0))
    timesT = jnp.transpose(times)
    wmapT = W_map.T
    srcT = pltpu.with_memory_space_constraint(srcT, pltpu.MemorySpace.HBM)

    ts_col, var_pe, eye128 = map(jnp.asarray, _np_tables(V))

    full = lambda shape: pl.BlockSpec(shape, lambda i: tuple(0 for _ in shape))
    operands = (
        srcT, timesT, static, ts_col, var_pe, eye128,
        W_value.reshape(1, 16), b_value.reshape(1, 16),
        wmapT, b_map.reshape(1, -1),
        W_emb, b_emb.reshape(1, -1),
        W_mlp1, b_mlp1.reshape(1, -1),
        W_mlp2, b_mlp2.reshape(1, -1),
    )
    in_specs = [
        pl.BlockSpec((B, 2 * V, TBT), lambda i: (0, 0, i)),
        pl.BlockSpec((B, TBT), lambda i: (0, i)),
    ] + [full(op.shape) for op in operands[2:]]

    return pl.pallas_call(
        _seft_body,
        grid=(grid,),
        in_specs=in_specs,
        out_specs=pl.BlockSpec((B, 2), lambda i: (0, 0)),
        out_shape=jax.ShapeDtypeStruct((B, 2), jnp.float32),
        scratch_shapes=[
            pltpu.VMEM((8 * D_PE, TBT), jnp.float32),
            pltpu.VMEM((8 * D_PE, TBT), jnp.float32),
            pltpu.VMEM((B, V, TBT), jnp.float32),
            pltpu.VMEM((B, V, TBT), jnp.float32),
        ],
        compiler_params=pltpu.CompilerParams(
            dimension_semantics=("arbitrary",)),
    )(*operands)


def kernel(src, static, times, lengths, W_value, b_value, W_map, b_map,
           W_emb, b_emb, W_mlp1, b_mlp1, W_mlp2, b_mlp2):
    del lengths
    return _seft(src, static, times, W_value, b_value, W_map, b_map,
                 W_emb, b_emb, W_mlp1, b_mlp1, W_mlp2, b_mlp2)


# speedup vs baseline: 4.2989x; 1.0251x over previous
import functools

import jax
import jax.numpy as jnp
import numpy as np
from jax.experimental import pallas as pl
from jax.experimental.pallas import tpu as pltpu

MAX_LEN = 2048
D_PE = 16
N_TS = D_PE // 2
HIGHEST = jax.lax.Precision.HIGHEST
DN_T = (((1,), (1,)), ((), ()))


def _np_tables(V):
    ts = (MAX_LEN ** np.linspace(0.0, 1.0, N_TS)).astype(np.float32)
    ts_col = np.repeat(ts, 16).reshape(N_TS * 16, 1).astype(np.float32)
    scaled = np.arange(V, dtype=np.float32)[:, None] / ts[None, :]
    var_pe = np.concatenate([np.sin(scaled), np.cos(scaled)], axis=1)
    eye128 = np.eye(128, dtype=np.float32)
    return ts_col, var_pe.astype(np.float32), eye128


def _seft_body(src_ref, times_ref, static_ref, tscol_ref, varpe_ref,
               eye_ref, wv_ref, bv_ref, wmapT_ref, bmap_ref,
               wemb_ref, bemb_ref, wm1_ref, bm1_ref, wm2_ref, bm2_ref,
               out_ref, acc_sin, acc_cos, colcnt_l, colfea_l):
    i = pl.program_id(0)
    f32 = jnp.float32

    @pl.when(i == 0)
    def _init():
        acc_sin[...] = jnp.zeros_like(acc_sin)
        acc_cos[...] = jnp.zeros_like(acc_cos)
        colcnt_l[...] = jnp.zeros_like(colcnt_l)
        colfea_l[...] = jnp.zeros_like(colfea_l)

    x = src_ref[...]
    v = x.shape[1] // 2
    fea = x[:, :v, :]
    mask = (fea != 0.0).astype(f32)
    colcnt_l[...] += mask
    colfea_l[...] += jnp.sum(fea, axis=1)
    rowcnt = jnp.sum(mask, axis=1)

    tb = times_ref[...]
    t_big = jnp.concatenate([tb] * N_TS, axis=0) / tscol_ref[...]
    rc8 = jnp.concatenate([rowcnt] * N_TS, axis=0)
    acc_sin[...] += jnp.sin(t_big) * rc8
    acc_cos[...] += jnp.cos(t_big) * rc8

    @pl.when(i == pl.num_programs(0) - 1)
    def _epilogue():
        cc = jnp.sum(colcnt_l[...], axis=2)
        count = jnp.sum(cc, axis=1, keepdims=True)
        sumfea = jnp.sum(colfea_l[...], axis=1, keepdims=True)
        denom = jnp.maximum(count, 1.0)
        sum_var = jnp.dot(cc, varpe_ref[...],
                          preferred_element_type=f32, precision=HIGHEST)

        acc2 = jnp.concatenate(
            [jnp.sum(acc_sin[...], axis=1, keepdims=True),
             jnp.sum(acc_cos[...], axis=1, keepdims=True)], axis=1)
        accr = jax.lax.dot_general(
            acc2, eye_ref[...], (((0,), (0,)), ((), ())),
            preferred_element_type=f32, precision=HIGHEST)
        rows = [accr[0:1, 16 * d:16 * (d + 1)] for d in range(N_TS)]
        rows += [accr[1:2, 16 * d:16 * (d + 1)] for d in range(N_TS)]
        spe_t = jnp.concatenate(rows, axis=0)

        wsum = wmapT_ref[0:3 * D_PE, :] + wmapT_ref[3 * D_PE:, :]
        w_pe = wsum[0:16, :]
        w_val = wsum[16:32, :]
        w_var = wsum[32:48, :]
        term_pe = jax.lax.dot_general(
            spe_t, w_pe, (((0,), (0,)), ((), ())),
            preferred_element_type=f32, precision=HIGHEST)
        sum_val = (jnp.dot(sumfea, wv_ref[...],
                           preferred_element_type=f32, precision=HIGHEST)
                   + jnp.dot(count, bv_ref[...],
                             preferred_element_type=f32, precision=HIGHEST))
        term_val = jnp.dot(sum_val, w_val,
                           preferred_element_type=f32, precision=HIGHEST)
        term_var = jnp.dot(sum_var, w_var,
                           preferred_element_type=f32, precision=HIGHEST)

        raw = term_pe + term_val + term_var
        out128 = jnp.where(count > 0, raw / denom, 0.0) + bmap_ref[...]
        emb = jax.lax.dot_general(
            static_ref[...], wemb_ref[...], DN_T,
            preferred_element_type=f32, precision=HIGHEST) + bemb_ref[...]
        cat = jnp.concatenate([out128, emb], axis=1)
        h = jnp.maximum(
            jax.lax.dot_general(cat, wm1_ref[...], DN_T,
                                preferred_element_type=f32,
                                precision=HIGHEST) + bm1_ref[...], 0.0)
        out_ref[...] = jax.lax.dot_general(
            h, wm2_ref[...], DN_T, preferred_element_type=f32,
            precision=HIGHEST) + bm2_ref[...]


@functools.partial(jax.jit, static_argnames=())
def _seft(src, static, times, W_value, b_value, W_map, b_map, W_emb, b_emb,
          W_mlp1, b_mlp1, W_mlp2, b_mlp2):
    T, B = src.shape[0], src.shape[1]
    V = src.shape[2] // 2
    TBT = 512
    grid = T // TBT

    srcT = jnp.transpose(src, (1, 2, 0))
    timesT = jnp.transpose(times)
    wmapT = W_map.T
    srcT = pltpu.with_memory_space_constraint(srcT, pltpu.MemorySpace.HBM)

    ts_col, var_pe, eye128 = map(jnp.asarray, _np_tables(V))

    full = lambda shape: pl.BlockSpec(shape, lambda i: tuple(0 for _ in shape))
    operands = (
        srcT, timesT, static, ts_col, var_pe, eye128,
        W_value.reshape(1, 16), b_value.reshape(1, 16),
        wmapT, b_map.reshape(1, -1),
        W_emb, b_emb.reshape(1, -1),
        W_mlp1, b_mlp1.reshape(1, -1),
        W_mlp2, b_mlp2.reshape(1, -1),
    )
    in_specs = [
        pl.BlockSpec((B, 2 * V, TBT), lambda i: (0, 0, i)),
        pl.BlockSpec((B, TBT), lambda i: (0, i)),
    ] + [full(op.shape) for op in operands[2:]]

    return pl.pallas_call(
        _seft_body,
        grid=(grid,),
        in_specs=in_specs,
        out_specs=pl.BlockSpec((B, 2), lambda i: (0, 0)),
        out_shape=jax.ShapeDtypeStruct((B, 2), jnp.float32),
        scratch_shapes=[
            pltpu.VMEM((8 * D_PE, TBT), jnp.float32),
            pltpu.VMEM((8 * D_PE, TBT), jnp.float32),
            pltpu.VMEM((B, V, TBT), jnp.float32),
            pltpu.VMEM((B, TBT), jnp.float32),
        ],
        compiler_params=pltpu.CompilerParams(
            dimension_semantics=("arbitrary",)),
    )(*operands)


def kernel(src, static, times, lengths, W_value, b_value, W_map, b_map,
           W_emb, b_emb, W_mlp1, b_mlp1, W_mlp2, b_mlp2):
    del lengths
    return _seft(src, static, times, W_value, b_value, W_map, b_map,
                 W_emb, b_emb, W_mlp1, b_mlp1, W_mlp2, b_mlp2)
